# baseline (speedup 1.0000x reference)
import functools

import jax
import jax.numpy as jnp
from jax import lax
from jax.experimental import pallas as pl
from jax.experimental.pallas import tpu as pltpu
from jax.experimental.pallas import tpu_sc as plsc

B = 512
NUM_AGENTS = 64
NUM_HIVES = 16
NUM_ENTITIES = 80
LANES = 16
NUM_WORKERS = 32
B_PER_W = B // NUM_WORKERS
A_UNROLL = 4


def _sc_vote_body(votes_hbm, hv_hbm, mf_hbm, term_hbm,
                  votes_v, hv_v, counts_v, mf_v, term_v, sem0, sem1):
    c = lax.axis_index("c")
    s = lax.axis_index("s")
    wid = s * 2 + c
    b0 = wid * B_PER_W

    bt = wid // 8
    bi0 = (wid % 8) * LANES

    cp0 = pltpu.make_async_copy(
        votes_hbm.at[:, :, bt, :, pl.ds(bi0, B_PER_W)], votes_v, sem0)
    cp1 = pltpu.make_async_copy(
        hv_hbm.at[:, :, pl.ds(b0, B_PER_W)], hv_v, sem1)
    cp0.start()
    cp1.start()

    lane = lax.iota(jnp.int32, LANES)
    zero16 = jnp.zeros((LANES,), jnp.float32)
    ones16 = jnp.ones((LANES,), jnp.float32)
    for h in range(NUM_HIVES):
        counts_v[h, :] = zero16

    cp0.wait()

    def half_scan(a, ht):
        best_val = votes_v[a, ht, 0, :]
        best_idx = jnp.full((LANES,), ht * 8, jnp.int32)
        for hi in range(1, 8):
            col = votes_v[a, ht, hi, :]
            m = col > best_val
            best_val = jnp.where(m, col, best_val)
            best_idx = jnp.where(
                m, jnp.full((LANES,), ht * 8 + hi, jnp.int32), best_idx)
        return best_val, best_idx

    def agent_body(i, carry):
        for u in range(A_UNROLL):
            a = i * A_UNROLL + u
            val_lo, idx_lo = half_scan(a, 0)
            val_hi, idx_hi = half_scan(a, 1)
            m = val_hi > val_lo
            best_idx = jnp.where(m, idx_hi, idx_lo)
            plsc.addupdate_scatter(counts_v, [best_idx, lane], ones16)
        return carry

    lax.fori_loop(0, NUM_AGENTS // A_UNROLL, agent_body, 0)
    cp1.wait()

    mf = counts_v[0, :]
    val = counts_v[0, :] * hv_v[0, 0, :]
    for h in range(1, NUM_HIVES):
        ch = counts_v[h, :]
        mf = jnp.maximum(mf, ch)
        val = val + ch * hv_v[h, 0, :]

    max_freq = mf * (1.0 / NUM_AGENTS)
    x = 30.0 * (max_freq - 0.7)
    term = val * (1.0 + jnp.exp(x)) * (1.0 / 100.0)

    mf_v[...] = max_freq
    term_v[...] = term
    pltpu.sync_copy(mf_v, mf_hbm.at[pl.ds(b0, B_PER_W)])
    pltpu.sync_copy(term_v, term_hbm.at[wid])


_sc_vote = functools.partial(
    pl.kernel,
    out_type=(jax.ShapeDtypeStruct((B,), jnp.float32),
              jax.ShapeDtypeStruct((NUM_WORKERS, LANES), jnp.float32)),
    mesh=plsc.VectorSubcoreMesh(core_axis_name="c", subcore_axis_name="s"),
    compiler_params=pltpu.CompilerParams(
        needs_layout_passes=False, use_tc_tiling_on_sc=False),
    scratch_types=[
        pltpu.VMEM((NUM_AGENTS, 2, 8, B_PER_W), jnp.float32),
        pltpu.VMEM((NUM_HIVES, 1, B_PER_W), jnp.float32),
        pltpu.VMEM((NUM_HIVES, B_PER_W), jnp.float32),
        pltpu.VMEM((LANES,), jnp.float32),
        pltpu.VMEM((LANES,), jnp.float32),
        pltpu.SemaphoreType.DMA,
        pltpu.SemaphoreType.DMA,
    ],
)(_sc_vote_body)


def _tc_mov_body(mv_ref, movsum_ref):
    mv = mv_ref[...]
    norms = jnp.sqrt(jnp.sum(mv * mv, axis=1))
    movsum_ref[...] = jnp.reshape(jnp.sum(norms), (1, 1))


def _tc_cost_body(term_ref, movsum_ref, cost_ref):
    cost_ref[...] = jnp.reshape(
        movsum_ref[0, 0] - jnp.sum(term_ref[...]), (1, 1))


def kernel(movements, utterances, votes, hive_values, locations):
    votes_t = jnp.transpose(votes, (1, 2, 0))
    votes_5d = votes_t.reshape(NUM_AGENTS, 2, 8, 4, 128).transpose(0, 1, 3, 2, 4)
    hv_t = jnp.transpose(hive_values, (1, 2, 0))
    mv_t = jnp.transpose(movements, (1, 2, 0))
    mf, terms = _sc_vote(votes_5d, hv_t)
    movsum = pl.pallas_call(
        _tc_mov_body,
        out_shape=jax.ShapeDtypeStruct((1, 1), jnp.float32),
    )(mv_t)
    cost = pl.pallas_call(
        _tc_cost_body,
        out_shape=jax.ShapeDtypeStruct((1, 1), jnp.float32),
    )(terms, movsum)
    return (cost[0, 0], mf)


# speedup vs baseline: 1.0347x; 1.0347x over previous
import functools

import jax
import jax.numpy as jnp
from jax import lax
from jax.experimental import pallas as pl
from jax.experimental.pallas import tpu as pltpu
from jax.experimental.pallas import tpu_sc as plsc

B = 512
NUM_AGENTS = 64
NUM_HIVES = 16
NUM_ENTITIES = 80
LANES = 16
NUM_WORKERS = 32
A_PER_W = NUM_AGENTS // NUM_WORKERS
NUM_BG = B // LANES


def _sc_vote_body(votes_hbm, cnt_hbm, votes_v, counts_v, sem0, sem1):
    c = lax.axis_index("c")
    s = lax.axis_index("s")
    wid = s * 2 + c
    a0 = wid * A_PER_W

    cp0 = pltpu.make_async_copy(votes_hbm.at[a0], votes_v.at[0], sem0)
    cp1 = pltpu.make_async_copy(votes_hbm.at[a0 + 1], votes_v.at[1], sem1)
    cp0.start()
    cp1.start()

    lane = lax.iota(jnp.int32, LANES)
    zero16 = jnp.zeros((LANES,), jnp.float32)
    ones16 = jnp.ones((LANES,), jnp.float32)

    def zero_body(j, carry):
        for h in range(NUM_HIVES):
            counts_v[h, pl.ds(j * LANES, LANES)] = zero16
        return carry

    lax.fori_loop(0, B // LANES, zero_body, 0)
    cp0.wait()
    cp1.wait()

    def half_scan(a, bg, h_lo):
        best_val = votes_v[a, h_lo, pl.ds(bg * LANES, LANES)]
        best_idx = jnp.full((LANES,), h_lo, jnp.int32)
        for h in range(h_lo + 1, h_lo + 8):
            col = votes_v[a, h, pl.ds(bg * LANES, LANES)]
            m = col > best_val
            best_val = jnp.where(m, col, best_val)
            best_idx = jnp.where(m, jnp.full((LANES,), h, jnp.int32), best_idx)
        return best_val, best_idx

    def bg_body(i, carry):
        for u in range(2):
            bg = i * 2 + u
            b_vec = bg * LANES + lane
            for a in range(A_PER_W):
                val_lo, idx_lo = half_scan(a, bg, 0)
                val_hi, idx_hi = half_scan(a, bg, 8)
                m = val_hi > val_lo
                best_idx = jnp.where(m, idx_hi, idx_lo)
                plsc.addupdate_scatter(counts_v, [best_idx, b_vec], ones16)
        return carry

    out_cps = []
    for q in range(4):
        lax.fori_loop(4 * q, 4 * q + 4, bg_body, 0)
        cp = pltpu.make_async_copy(
            counts_v.at[:, pl.ds(q * 128, 128)],
            cnt_hbm.at[pl.ds(wid * NUM_HIVES, NUM_HIVES), pl.ds(q * 128, 128)],
            sem0)
        cp.start()
        out_cps.append(cp)
    for cp in out_cps:
        cp.wait()


_sc_vote = functools.partial(
    pl.kernel,
    out_type=jax.ShapeDtypeStruct((NUM_WORKERS * NUM_HIVES, B), jnp.float32),
    mesh=plsc.VectorSubcoreMesh(core_axis_name="c", subcore_axis_name="s"),
    compiler_params=pltpu.CompilerParams(
        needs_layout_passes=False,
        disable_bounds_checks=True,
        disable_semaphore_checks=True,
    ),
    scratch_types=[
        pltpu.VMEM((A_PER_W, NUM_HIVES, B), jnp.float32),
        pltpu.VMEM((NUM_HIVES, B), jnp.float32),
        pltpu.SemaphoreType.DMA,
        pltpu.SemaphoreType.DMA,
    ],
)(_sc_vote_body)


def _tc_mov_body(mv_ref, movsum_ref):
    mv = mv_ref[...]
    norms = jnp.sqrt(jnp.sum(mv * mv, axis=1))
    movsum_ref[...] = jnp.reshape(jnp.sum(norms), (1, 1))


def _tc_cost_body(cnt_ref, hv_ref, movsum_ref, cost_ref, mf_ref):
    parts = cnt_ref[...].reshape(NUM_WORKERS, NUM_HIVES, B)
    counts = jnp.sum(parts, axis=0)
    mf = jnp.max(counts, axis=0) * (1.0 / NUM_AGENTS)
    val = jnp.sum(counts * hv_ref[:, 0, :], axis=0)
    term = val * (1.0 + jnp.exp(30.0 * (mf - 0.7))) * (1.0 / 100.0)

    cost_ref[...] = jnp.reshape(movsum_ref[0, 0] - jnp.sum(term), (1, 1))
    mf_ref[...] = jnp.reshape(mf, (1, B))


def kernel(movements, utterances, votes, hive_values, locations):
    votes_t = jnp.transpose(votes, (1, 2, 0))
    parts = _sc_vote(votes_t)
    hv_t = jnp.transpose(hive_values, (1, 2, 0))
    mv_t = jnp.transpose(movements, (1, 2, 0))
    movsum = pl.pallas_call(
        _tc_mov_body,
        out_shape=jax.ShapeDtypeStruct((1, 1), jnp.float32),
    )(mv_t)
    cost, mf = pl.pallas_call(
        _tc_cost_body,
        out_shape=(jax.ShapeDtypeStruct((1, 1), jnp.float32),
                   jax.ShapeDtypeStruct((1, B), jnp.float32)),
    )(parts, hv_t, movsum)
    return (cost[0, 0], mf.reshape(B))


# speedup vs baseline: 1.0467x; 1.0116x over previous
import functools

import jax
import jax.numpy as jnp
from jax import lax
from jax.experimental import pallas as pl
from jax.experimental.pallas import tpu as pltpu
from jax.experimental.pallas import tpu_sc as plsc

B = 512
NUM_AGENTS = 64
NUM_HIVES = 16
NUM_ENTITIES = 80
LANES = 16
NUM_WORKERS = 32
A_PER_W = NUM_AGENTS // NUM_WORKERS
NUM_BG = B // LANES


def _sc_vote_body(votes_hbm, cnt_hbm, votes_v, counts_v, sem0, sem1):
    c = lax.axis_index("c")
    s = lax.axis_index("s")
    wid = s * 2 + c
    a0 = wid * A_PER_W

    cp0 = pltpu.make_async_copy(votes_hbm.at[a0], votes_v.at[0], sem0)
    cp1 = pltpu.make_async_copy(votes_hbm.at[a0 + 1], votes_v.at[1], sem1)
    cp0.start()
    cp1.start()

    lane = lax.iota(jnp.int32, LANES)
    zero16 = jnp.zeros((LANES,), jnp.float32)
    ones16 = jnp.ones((LANES,), jnp.float32)

    def zero_body(j, carry):
        for h in range(NUM_HIVES):
            counts_v[h, pl.ds(j * LANES, LANES)] = zero16
        return carry

    lax.fori_loop(0, B // LANES, zero_body, 0)
    cp0.wait()
    cp1.wait()

    def half_scan(a, bg, h_lo):
        best_val = votes_v[a, h_lo, pl.ds(bg * LANES, LANES)]
        best_idx = jnp.full((LANES,), h_lo, jnp.int32)
        for h in range(h_lo + 1, h_lo + 8):
            col = votes_v[a, h, pl.ds(bg * LANES, LANES)]
            m = col > best_val
            best_val = jnp.where(m, col, best_val)
            best_idx = jnp.where(m, jnp.full((LANES,), h, jnp.int32), best_idx)
        return best_val, best_idx

    def bg_body(i, carry):
        for u in range(4):
            bg = i * 4 + u
            b_vec = bg * LANES + lane
            for a in range(A_PER_W):
                val_lo, idx_lo = half_scan(a, bg, 0)
                val_hi, idx_hi = half_scan(a, bg, 8)
                m = val_hi > val_lo
                best_idx = jnp.where(m, idx_hi, idx_lo)
                plsc.addupdate_scatter(counts_v, [best_idx, b_vec], ones16)
        return carry

    lax.fori_loop(0, NUM_BG // 4, bg_body, 0)

    pltpu.sync_copy(counts_v, cnt_hbm.at[pl.ds(wid * NUM_HIVES, NUM_HIVES)])


_sc_vote = functools.partial(
    pl.kernel,
    out_type=jax.ShapeDtypeStruct((NUM_WORKERS * NUM_HIVES, B), jnp.float32),
    mesh=plsc.VectorSubcoreMesh(core_axis_name="c", subcore_axis_name="s"),
    compiler_params=pltpu.CompilerParams(
        needs_layout_passes=False,
    ),
    scratch_types=[
        pltpu.VMEM((A_PER_W, NUM_HIVES, B), jnp.float32),
        pltpu.VMEM((NUM_HIVES, B), jnp.float32),
        pltpu.SemaphoreType.DMA,
        pltpu.SemaphoreType.DMA,
    ],
)(_sc_vote_body)


def _tc_mov_body(mv_ref, movsum_ref):
    mv = mv_ref[...]
    norms = jnp.sqrt(jnp.sum(mv * mv, axis=1))
    movsum_ref[...] = jnp.reshape(jnp.sum(norms), (1, 1))


def _tc_cost_body(cnt_ref, hv_ref, movsum_ref, cost_ref, mf_ref):
    parts = cnt_ref[...].reshape(NUM_WORKERS, NUM_HIVES, B)
    counts = jnp.sum(parts, axis=0)
    mf = jnp.max(counts, axis=0) * (1.0 / NUM_AGENTS)
    val = jnp.sum(counts * hv_ref[:, 0, :], axis=0)
    term = val * (1.0 + jnp.exp(30.0 * (mf - 0.7))) * (1.0 / 100.0)

    cost_ref[...] = jnp.reshape(movsum_ref[0, 0] - jnp.sum(term), (1, 1))
    mf_ref[...] = jnp.reshape(mf, (1, B))


def kernel(movements, utterances, votes, hive_values, locations):
    votes_t = jnp.transpose(votes, (1, 2, 0))
    parts = _sc_vote(votes_t)
    hv_t = jnp.transpose(hive_values, (1, 2, 0))
    mv_t = jnp.transpose(movements, (1, 2, 0))
    movsum = pl.pallas_call(
        _tc_mov_body,
        out_shape=jax.ShapeDtypeStruct((1, 1), jnp.float32),
    )(mv_t)
    cost, mf = pl.pallas_call(
        _tc_cost_body,
        out_shape=(jax.ShapeDtypeStruct((1, 1), jnp.float32),
                   jax.ShapeDtypeStruct((1, B), jnp.float32)),
    )(parts, hv_t, movsum)
    return (cost[0, 0], mf.reshape(B))


# speedup vs baseline: 1.0646x; 1.0171x over previous
import functools

import jax
import jax.numpy as jnp
from jax import lax
from jax.experimental import pallas as pl
from jax.experimental.pallas import tpu as pltpu
from jax.experimental.pallas import tpu_sc as plsc

B = 512
NUM_AGENTS = 64
NUM_HIVES = 16
NUM_ENTITIES = 80
LANES = 16
NUM_WORKERS = 32
A_PER_W = NUM_AGENTS // NUM_WORKERS
NUM_BG = B // LANES


def _sc_vote_body(votes_hbm, cnt_hbm, votes_v, counts_v, sem0, sem1):
    c = lax.axis_index("c")
    s = lax.axis_index("s")
    wid = s * 2 + c
    a0 = wid * A_PER_W

    cp0 = pltpu.make_async_copy(votes_hbm.at[a0], votes_v.at[0], sem0)
    cp1 = pltpu.make_async_copy(votes_hbm.at[a0 + 1], votes_v.at[1], sem1)
    cp0.start()
    cp1.start()

    lane = lax.iota(jnp.int32, LANES)
    zero16 = jnp.zeros((LANES,), jnp.float32)
    ones16 = jnp.ones((LANES,), jnp.float32)

    def zero_body(j, carry):
        for h in range(NUM_HIVES):
            counts_v[h, pl.ds(j * LANES, LANES)] = zero16
        return carry

    lax.fori_loop(0, B // LANES, zero_body, 0)
    cp0.wait()
    cp1.wait()

    def half_scan(a, bg, h_lo):
        best_val = votes_v[a, h_lo, pl.ds(bg * LANES, LANES)]
        best_idx = jnp.full((LANES,), h_lo, jnp.int32)
        for h in range(h_lo + 1, h_lo + 8):
            col = votes_v[a, h, pl.ds(bg * LANES, LANES)]
            m = col > best_val
            best_val = jnp.where(m, col, best_val)
            best_idx = jnp.where(m, jnp.full((LANES,), h, jnp.int32), best_idx)
        return best_val, best_idx

    def bg_body(i, carry):
        for u in range(2):
            bg = i * 2 + u
            b_vec = bg * LANES + lane
            for a in range(A_PER_W):
                val_lo, idx_lo = half_scan(a, bg, 0)
                val_hi, idx_hi = half_scan(a, bg, 8)
                m = val_hi > val_lo
                best_idx = jnp.where(m, idx_hi, idx_lo)
                plsc.addupdate_scatter(counts_v, [best_idx, b_vec], ones16)
        return carry

    lax.fori_loop(0, NUM_BG // 2, bg_body, 0)

    pltpu.sync_copy(counts_v, cnt_hbm.at[pl.ds(wid * NUM_HIVES, NUM_HIVES)])


_sc_vote = functools.partial(
    pl.kernel,
    out_type=jax.ShapeDtypeStruct((NUM_WORKERS * NUM_HIVES, B), jnp.float32),
    mesh=plsc.VectorSubcoreMesh(core_axis_name="c", subcore_axis_name="s"),
    compiler_params=pltpu.CompilerParams(needs_layout_passes=False),
    scratch_types=[
        pltpu.VMEM((A_PER_W, NUM_HIVES, B), jnp.float32),
        pltpu.VMEM((NUM_HIVES, B), jnp.float32),
        pltpu.SemaphoreType.DMA,
        pltpu.SemaphoreType.DMA,
    ],
)(_sc_vote_body)


def _tc_mov_body(mv_ref, movsum_ref):
    mv = mv_ref[...]
    norms = jnp.sqrt(jnp.sum(mv * mv, axis=1))
    movsum_ref[...] = jnp.reshape(jnp.sum(norms), (1, 1))


def _tc_cost_body(cnt_ref, hv_ref, movsum_ref, cost_ref, mf_ref):
    parts = cnt_ref[...].reshape(NUM_WORKERS, NUM_HIVES, B)
    counts = jnp.sum(parts, axis=0)
    mf = jnp.max(counts, axis=0) * (1.0 / NUM_AGENTS)
    val = jnp.sum(counts * hv_ref[:, 0, :], axis=0)
    term = val * (1.0 + jnp.exp(30.0 * (mf - 0.7))) * (1.0 / 100.0)

    cost_ref[...] = jnp.reshape(movsum_ref[0, 0] - jnp.sum(term), (1, 1))
    mf_ref[...] = jnp.reshape(mf, (1, B))


def kernel(movements, utterances, votes, hive_values, locations):
    votes_t = jnp.transpose(votes, (1, 2, 0))
    parts = _sc_vote(votes_t)
    hv_t = jnp.transpose(hive_values, (1, 2, 0))
    mv_t = jnp.transpose(movements, (1, 2, 0))
    movsum = pl.pallas_call(
        _tc_mov_body,
        out_shape=jax.ShapeDtypeStruct((1, 1), jnp.float32),
    )(mv_t)
    cost, mf = pl.pallas_call(
        _tc_cost_body,
        out_shape=(jax.ShapeDtypeStruct((1, 1), jnp.float32),
                   jax.ShapeDtypeStruct((1, B), jnp.float32)),
    )(parts, hv_t, movsum)
    return (cost[0, 0], mf.reshape(B))
